# baseline (speedup 1.0000x reference)
import functools

import jax
import jax.numpy as jnp
from jax import lax
from jax.experimental import pallas as pl
from jax.experimental.pallas import tpu as pltpu
from jax.experimental.pallas import tpu_sc as plsc

_VOCAB = 100000
_D = 64
_S = 8
_B = 4096
_SEQ = 52
_SEQP = 56
_L = 16
_NC = 2
_NS = 16
_NW = _NC * _NS
_BPW = _B // _NW
_KD = _D // _L
_NBUF = 3
_WSC = 32
_NQ = _BPW // _WSC


@functools.partial(
    pl.kernel,
    mesh=plsc.VectorSubcoreMesh(core_axis_name="c", subcore_axis_name="s"),
    compiler_params=pltpu.CompilerParams(use_tc_tiling_on_sc=True,
                                         needs_layout_passes=False),
    out_type=jax.ShapeDtypeStruct((_NW, _S, 8 * _L), jnp.float32),
    scratch_types=[
        pltpu.VMEM((_SEQP, _BPW), jnp.int32),
        pltpu.VMEM((_NBUF, _BPW, 8 * _L), jnp.float32),
        pltpu.VMEM((_BPW, _D), jnp.float32),
        pltpu.VMEM((2, _WSC, _S * _D), jnp.float32),
        pltpu.VMEM((_S, 8 * _L), jnp.float32),
        pltpu.SemaphoreType.DMA,
        pltpu.SemaphoreType.DMA,
        pltpu.SemaphoreType.DMA,
        pltpu.SemaphoreType.DMA,
        pltpu.VMEM((_BPW, _SEQ), jnp.int32),
    ],
)
def _sense_partials(x_hbm, wgp_hbm, ws2_hbm, out_hbm,
                    x_v, rows_v, acc_v, ws_v, part_v,
                    sem_ws, sem0, sem1, sem2, xr_v):
    wid = lax.axis_index("s") * _NC + lax.axis_index("c")
    sems = (sem0, sem1, sem2)
    zeros = jnp.zeros((_L,), jnp.float32)
    iota = lax.iota(jnp.int32, _L)

    pltpu.sync_copy(x_hbm.at[pl.ds(wid * _BPW, _BPW)], xr_v)
    for c in range(_SEQ):
        cv = jnp.full((_L,), c, jnp.int32)
        for j in range(_BPW // _L):
            rows = jnp.full((_L,), j * _L, jnp.int32) + iota
            x_v[c, pl.ds(j * _L, _L)] = plsc.load_gather(xr_v, [rows, cv])

    for q in range(2):
        pltpu.async_copy(ws2_hbm.at[x_v.at[0, pl.ds(q * _WSC, _WSC)]],
                         ws_v.at[q], sem_ws)

    def start_col(c, buf):
        pltpu.async_copy(wgp_hbm.at[x_v.at[c]], rows_v.at[buf], sems[buf])

    def wait_col(c, buf):
        pltpu.make_async_copy(
            wgp_hbm.at[x_v.at[c]], rows_v.at[buf], sems[buf]).wait()

    def acc_col(buf, first):
        def body(i, carry):
            for k in range(_KD):
                sl = pl.ds(k * _L, _L)
                v = rows_v[buf, i, sl]
                if first:
                    acc_v[i, sl] = v
                else:
                    plsc.addupdate(acc_v.at[i, sl], v)
            return carry
        lax.fori_loop(0, _BPW, body, 0, unroll=4)

    for t in range(_NBUF):
        start_col(2 + t, t)

    wait_col(2, 0)
    acc_col(0, first=True)
    start_col(5, 0)

    def ring_body(j, carry):
        c0 = 3 + 3 * j
        for t in range(3):
            buf = (1 + t) % _NBUF
            c = c0 + t
            wait_col(c, buf)
            acc_col(buf, first=False)

            @pl.when(c + _NBUF < _SEQ)
            def _():
                start_col(c + _NBUF, buf)
        return carry

    lax.fori_loop(0, 16, ring_body, 0)

    wait_col(51, 1)
    acc_col(1, first=False)

    accs = tuple(zeros for _ in range(_S))
    for q in range(_NQ):
        qb = q % 2
        idxref = x_v.at[0, pl.ds(q * _WSC, _WSC)]
        pltpu.make_async_copy(ws2_hbm.at[idxref], ws_v.at[qb], sem_ws).wait()

        def score_body(i, acc_c, q=q, qb=qb):
            ctx = [acc_v[q * _WSC + i, pl.ds(k * _L, _L)]
                   for k in range(_KD)]
            out = []
            for s in range(_S):
                a = acc_c[s]
                for k in range(_KD):
                    a = a + ws_v[qb, i, pl.ds(s * _D + k * _L, _L)] * ctx[k]
                out.append(a)
            return tuple(out)

        accs = lax.fori_loop(0, _WSC, score_body, accs)

        if q + 2 < _NQ:
            nidx = x_v.at[0, pl.ds((q + 2) * _WSC, _WSC)]
            pltpu.async_copy(ws2_hbm.at[nidx], ws_v.at[qb], sem_ws)

    for s in range(_S):
        for k in range(8):
            part_v[s, pl.ds(k * _L, _L)] = accs[s] if k == 0 else zeros
    pltpu.sync_copy(part_v, out_hbm.at[wid])


_PBLK = 2000


def _pad_body(wg_ref, out_ref):
    blk = wg_ref[...]
    out_ref[...] = jnp.concatenate(
        [blk, jnp.zeros_like(blk)], axis=1)


_pad_wg = pl.pallas_call(
    _pad_body,
    grid=(_VOCAB // _PBLK,),
    in_specs=[pl.BlockSpec((_PBLK, _D), lambda i: (i, 0))],
    out_specs=pl.BlockSpec((_PBLK, 2 * _D), lambda i: (i, 0)),
    out_shape=jax.ShapeDtypeStruct((_VOCAB, 2 * _D), jnp.float32),
)


@jax.jit
def kernel(x, W_g, W_s):
    wgp = _pad_wg(W_g)
    ws2 = W_s.reshape(_VOCAB, _S * _D)
    partials = _sense_partials(x, wgp, ws2)
    return jax.nn.sigmoid(jnp.sum(partials, axis=(0, 2)))


# speedup vs baseline: 1.0801x; 1.0801x over previous
import functools

import jax
import jax.numpy as jnp
from jax import lax
from jax.experimental import pallas as pl
from jax.experimental.pallas import tpu as pltpu
from jax.experimental.pallas import tpu_sc as plsc

_VOCAB = 100000
_D = 64
_S = 8
_B = 4096
_SEQ = 52
_SEQP = 56
_L = 16
_NC = 2
_NS = 16
_NW = _NC * _NS
_BPW = _B // _NW
_KD = _D // _L
_NBUF = 3
_WSC = 32
_NQ = _BPW // _WSC


@functools.partial(
    pl.kernel,
    mesh=plsc.VectorSubcoreMesh(core_axis_name="c", subcore_axis_name="s"),
    compiler_params=pltpu.CompilerParams(use_tc_tiling_on_sc=True,
                                         needs_layout_passes=False),
    out_type=jax.ShapeDtypeStruct((_NW, _S, 8 * _L), jnp.float32),
    scratch_types=[
        pltpu.VMEM((_SEQP, _BPW), jnp.int32),
        pltpu.VMEM((_NBUF, _BPW, 8 * _L), jnp.float32),
        pltpu.VMEM((_BPW, _D), jnp.float32),
        pltpu.VMEM((2, _WSC, _S * _D), jnp.float32),
        pltpu.VMEM((_S, 8 * _L), jnp.float32),
        pltpu.SemaphoreType.DMA,
        pltpu.SemaphoreType.DMA,
        pltpu.SemaphoreType.DMA,
        pltpu.SemaphoreType.DMA,
        pltpu.VMEM((_BPW, _SEQ), jnp.int32),
    ],
)
def _sense_partials(x_hbm, wgp_hbm, ws2_hbm, out_hbm,
                    x_v, rows_v, acc_v, ws_v, part_v,
                    sem_ws, sem0, sem1, sem2, xr_v):
    wid = lax.axis_index("s") * _NC + lax.axis_index("c")
    sems = (sem0, sem1, sem2)
    zeros = jnp.zeros((_L,), jnp.float32)
    iota = lax.iota(jnp.int32, _L)

    pltpu.sync_copy(x_hbm.at[pl.ds(wid * _BPW, _BPW)], xr_v)
    for c in range(_SEQ):
        cv = jnp.full((_L,), c, jnp.int32)
        for j in range(_BPW // _L):
            rows = jnp.full((_L,), j * _L, jnp.int32) + iota
            x_v[c, pl.ds(j * _L, _L)] = plsc.load_gather(xr_v, [rows, cv])

    for q in range(2):
        pltpu.async_copy(ws2_hbm.at[x_v.at[0, pl.ds(q * _WSC, _WSC)]],
                         ws_v.at[q], sem_ws)

    def start_col(c, buf):
        pltpu.async_copy(wgp_hbm.at[x_v.at[c]], rows_v.at[buf], sems[buf])

    def wait_col(c, buf):
        pltpu.make_async_copy(
            wgp_hbm.at[x_v.at[c]], rows_v.at[buf], sems[buf]).wait()

    def acc_col(buf, first):
        def body(i, carry):
            for k in range(_KD):
                sl = pl.ds(k * _L, _L)
                v = rows_v[buf, i, sl]
                if first:
                    acc_v[i, sl] = v
                else:
                    plsc.addupdate(acc_v.at[i, sl], v)
            return carry
        lax.fori_loop(0, _BPW, body, 0, unroll=4)

    for t in range(_NBUF):
        start_col(2 + t, t)

    wait_col(2, 0)
    acc_col(0, first=True)
    start_col(5, 0)

    def ring_body(j, carry):
        c0 = 3 + 3 * j
        for t in range(3):
            buf = (1 + t) % _NBUF
            c = c0 + t
            wait_col(c, buf)
            acc_col(buf, first=False)

            @pl.when(c + _NBUF < _SEQ)
            def _():
                start_col(c + _NBUF, buf)
        return carry

    lax.fori_loop(0, 16, ring_body, 0)

    wait_col(51, 1)
    acc_col(1, first=False)

    accs = tuple(zeros for _ in range(_S))
    for q in range(_NQ):
        qb = q % 2
        idxref = x_v.at[0, pl.ds(q * _WSC, _WSC)]
        pltpu.make_async_copy(ws2_hbm.at[idxref], ws_v.at[qb], sem_ws).wait()

        def score_body(i, acc_c, q=q, qb=qb):
            ctx = [acc_v[q * _WSC + i, pl.ds(k * _L, _L)]
                   for k in range(_KD)]
            out = []
            for s in range(_S):
                a = acc_c[s]
                for k in range(_KD):
                    a = a + ws_v[qb, i, pl.ds(s * _D + k * _L, _L)] * ctx[k]
                out.append(a)
            return tuple(out)

        accs = lax.fori_loop(0, _WSC, score_body, accs)

        if q + 2 < _NQ:
            nidx = x_v.at[0, pl.ds((q + 2) * _WSC, _WSC)]
            pltpu.async_copy(ws2_hbm.at[nidx], ws_v.at[qb], sem_ws)

    for s in range(_S):
        for k in range(8):
            part_v[s, pl.ds(k * _L, _L)] = accs[s] if k == 0 else zeros
    pltpu.sync_copy(part_v, out_hbm.at[wid])


_PBLK = 2000


def _pad_body(wg_ref, out_ref):
    blk = wg_ref[...]
    out_ref[...] = jnp.concatenate(
        [blk, jnp.zeros_like(blk)], axis=1)


_pad_wg = pl.pallas_call(
    _pad_body,
    grid=(_VOCAB // _PBLK,),
    in_specs=[pl.BlockSpec((_PBLK, _D), lambda i: (i, 0))],
    out_specs=pl.BlockSpec((_PBLK, 2 * _D), lambda i: (i, 0)),
    out_shape=jax.ShapeDtypeStruct((_VOCAB, 2 * _D), jnp.float32),
)


@jax.jit
def kernel(x, W_g, W_s):
    wgp = jnp.concatenate([W_g, W_g], axis=1)
    ws2 = W_s.reshape(_VOCAB, _S * _D)
    partials = _sense_partials(x, wgp, ws2)
    return jax.nn.sigmoid(jnp.sum(partials, axis=(0, 2)))


# speedup vs baseline: 1.0888x; 1.0080x over previous
import functools

import jax
import jax.numpy as jnp
from jax import lax
from jax.experimental import pallas as pl
from jax.experimental.pallas import tpu as pltpu
from jax.experimental.pallas import tpu_sc as plsc

_VOCAB = 100000
_D = 64
_S = 8
_B = 4096
_SEQ = 52
_SEQP = 56
_L = 16
_NC = 2
_NS = 16
_NW = _NC * _NS
_BPW = _B // _NW
_KD = _D // _L
_NBUF = 3
_WSC = 32
_NQ = _BPW // _WSC


@functools.partial(
    pl.kernel,
    mesh=plsc.VectorSubcoreMesh(core_axis_name="c", subcore_axis_name="s"),
    compiler_params=pltpu.CompilerParams(use_tc_tiling_on_sc=True,
                                         needs_layout_passes=False),
    out_type=jax.ShapeDtypeStruct((_NW, _S, 8 * _L), jnp.float32),
    scratch_types=[
        pltpu.VMEM((_SEQP, _BPW), jnp.int32),
        pltpu.VMEM((_NBUF, _BPW, 8 * _L), jnp.float32),
        pltpu.VMEM((_BPW, _D), jnp.float32),
        pltpu.VMEM((2, _WSC, _S * _D), jnp.float32),
        pltpu.VMEM((_S, 8 * _L), jnp.float32),
        pltpu.SemaphoreType.DMA,
        pltpu.SemaphoreType.DMA,
        pltpu.SemaphoreType.DMA,
        pltpu.SemaphoreType.DMA,
        pltpu.VMEM((_BPW, _SEQ), jnp.int32),
    ],
)
def _sense_partials(x_hbm, wgp_hbm, ws2_hbm, out_hbm,
                    x_v, rows_v, acc_v, ws_v, part_v,
                    sem_ws, sem0, sem1, sem2, xr_v):
    wid = lax.axis_index("s") * _NC + lax.axis_index("c")
    sems = (sem0, sem1, sem2)
    zeros = jnp.zeros((_L,), jnp.float32)
    iota = lax.iota(jnp.int32, _L)

    pltpu.sync_copy(x_hbm.at[pl.ds(wid * _BPW, _BPW)], xr_v)
    for c in range(_SEQ):
        cv = jnp.full((_L,), c, jnp.int32)
        for j in range(_BPW // _L):
            rows = jnp.full((_L,), j * _L, jnp.int32) + iota
            x_v[c, pl.ds(j * _L, _L)] = plsc.load_gather(xr_v, [rows, cv])

    for q in range(2):
        pltpu.async_copy(ws2_hbm.at[x_v.at[0, pl.ds(q * _WSC, _WSC)]],
                         ws_v.at[q], sem_ws)

    def start_col(c, buf):
        pltpu.async_copy(wgp_hbm.at[x_v.at[c]], rows_v.at[buf], sems[buf])

    def wait_col(c, buf):
        pltpu.make_async_copy(
            wgp_hbm.at[x_v.at[c]], rows_v.at[buf], sems[buf]).wait()

    def acc_col(buf, first):
        def body(i, carry):
            for k in range(_KD):
                sl = pl.ds(k * _L, _L)
                v = rows_v[buf, i, sl]
                if first:
                    acc_v[i, sl] = v
                else:
                    plsc.addupdate(acc_v.at[i, sl], v)
            return carry
        lax.fori_loop(0, _BPW, body, 0, unroll=4)

    for t in range(_NBUF):
        start_col(2 + t, t)

    wait_col(2, 0)
    acc_col(0, first=True)
    start_col(5, 0)

    def ring_body(j, carry):
        c0 = 3 + 3 * j
        for t in range(3):
            buf = (1 + t) % _NBUF
            c = c0 + t
            wait_col(c, buf)
            acc_col(buf, first=False)

            @pl.when(c + _NBUF < _SEQ)
            def _():
                start_col(c + _NBUF, buf)
        return carry

    lax.fori_loop(0, 16, ring_body, 0)

    wait_col(51, 1)
    acc_col(1, first=False)

    accs = tuple(zeros for _ in range(_S))
    for q in range(_NQ):
        qb = q % 2
        idxref = x_v.at[0, pl.ds(q * _WSC, _WSC)]
        pltpu.make_async_copy(ws2_hbm.at[idxref], ws_v.at[qb], sem_ws).wait()

        def score_body(i, acc_c, q=q, qb=qb):
            ctx = [acc_v[q * _WSC + i, pl.ds(k * _L, _L)]
                   for k in range(_KD)]
            out = []
            for s in range(_S):
                a = acc_c[s]
                for k in range(_KD):
                    a = a + ws_v[qb, i, pl.ds(s * _D + k * _L, _L)] * ctx[k]
                out.append(a)
            return tuple(out)

        accs = lax.fori_loop(0, _WSC, score_body, accs)

        if q + 2 < _NQ:
            nidx = x_v.at[0, pl.ds((q + 2) * _WSC, _WSC)]
            pltpu.async_copy(ws2_hbm.at[nidx], ws_v.at[qb], sem_ws)

    for s in range(_S):
        for k in range(8):
            part_v[s, pl.ds(k * _L, _L)] = accs[s] if k == 0 else zeros
    pltpu.sync_copy(part_v, out_hbm.at[wid])


_PB = 128
_NPB = _VOCAB // _PB
_PTAIL = _VOCAB - _NPB * _PB
_KPB = -(-_NPB // _NW)


@functools.partial(
    pl.kernel,
    mesh=plsc.VectorSubcoreMesh(core_axis_name="c", subcore_axis_name="s"),
    compiler_params=pltpu.CompilerParams(use_tc_tiling_on_sc=True,
                                         needs_layout_passes=False),
    out_type=jax.ShapeDtypeStruct((_VOCAB, 2 * _D), jnp.float32),
    scratch_types=[
        pltpu.VMEM((2, _PB, _D), jnp.float32),
        pltpu.VMEM((2, _PB, 2 * _D), jnp.float32),
        pltpu.SemaphoreType.DMA,
        pltpu.SemaphoreType.DMA,
        pltpu.SemaphoreType.DMA,
    ],
)
def _pad_wg_sc(wg_hbm, out_hbm, in_v, st_v, semi0, semi1, semo):
    wid = lax.axis_index("s") * _NC + lax.axis_index("c")
    semis = (semi0, semi1)

    def start_in(k, t):
        b = wid + _NW * k

        @pl.when(b < _NPB)
        def _():
            pltpu.async_copy(wg_hbm.at[pl.ds(b * _PB, _PB)],
                             in_v.at[t], semis[t])

    def step(k, t):
        b = wid + _NW * k

        @pl.when(b < _NPB)
        def _():
            pltpu.make_async_copy(wg_hbm.at[pl.ds(b * _PB, _PB)],
                                  in_v.at[t], semis[t]).wait()

            def rb(i, carry):
                for kk in range(_KD):
                    sl = pl.ds(kk * _L, _L)
                    st_v[t, i, sl] = in_v[t, i, sl]
                return carry

            lax.fori_loop(0, _PB, rb, 0, unroll=4)
            pltpu.async_copy(st_v.at[t], out_hbm.at[pl.ds(b * _PB, _PB)],
                             semo)
            pltpu.make_async_copy(st_v.at[t],
                                  out_hbm.at[pl.ds(b * _PB, _PB)],
                                  semo).wait()

    start_in(0, 0)
    start_in(1, 1)

    def loop(j, carry):
        for t in range(2):
            k = 2 * j + t
            step(k, t)
            start_in(k + 2, t)
        return carry

    lax.fori_loop(0, (_KPB + 1) // 2, loop, 0)

    @pl.when(wid == _NW - 1)
    def _():
        r0 = _NPB * _PB
        pltpu.sync_copy(wg_hbm.at[pl.ds(r0, _PTAIL)],
                        in_v.at[0, pl.ds(0, _PTAIL)])

        def rb(i, carry):
            for kk in range(_KD):
                sl = pl.ds(kk * _L, _L)
                st_v[0, i, sl] = in_v[0, i, sl]
            return carry

        lax.fori_loop(0, _PTAIL, rb, 0, unroll=4)
        pltpu.sync_copy(st_v.at[0, pl.ds(0, _PTAIL)],
                        out_hbm.at[pl.ds(r0, _PTAIL)])


@jax.jit
def kernel(x, W_g, W_s):
    wgp = _pad_wg_sc(W_g)
    ws2 = W_s.reshape(_VOCAB, _S * _D)
    partials = _sense_partials(x, wgp, ws2)
    return jax.nn.sigmoid(jnp.sum(partials, axis=(0, 2)))


# speedup vs baseline: 1.0897x; 1.0008x over previous
import functools

import jax
import jax.numpy as jnp
from jax import lax
from jax.experimental import pallas as pl
from jax.experimental.pallas import tpu as pltpu
from jax.experimental.pallas import tpu_sc as plsc

_VOCAB = 100000
_D = 64
_S = 8
_B = 4096
_SEQ = 52
_SEQP = 56
_L = 16
_NC = 2
_NS = 16
_NW = _NC * _NS
_BPW = _B // _NW
_KD = _D // _L
_NBUF = 3
_WSC = 32
_NQ = _BPW // _WSC


@functools.partial(
    pl.kernel,
    mesh=plsc.VectorSubcoreMesh(core_axis_name="c", subcore_axis_name="s"),
    compiler_params=pltpu.CompilerParams(use_tc_tiling_on_sc=True,
                                         needs_layout_passes=False,
                                         skip_device_barrier=True),
    out_type=jax.ShapeDtypeStruct((_NW, _S, 8 * _L), jnp.float32),
    scratch_types=[
        pltpu.VMEM((_SEQP, _BPW), jnp.int32),
        pltpu.VMEM((_NBUF, _BPW, 8 * _L), jnp.float32),
        pltpu.VMEM((_BPW, _D), jnp.float32),
        pltpu.VMEM((2, _WSC, _S * _D), jnp.float32),
        pltpu.VMEM((_S, 8 * _L), jnp.float32),
        pltpu.SemaphoreType.DMA,
        pltpu.SemaphoreType.DMA,
        pltpu.SemaphoreType.DMA,
        pltpu.SemaphoreType.DMA,
        pltpu.VMEM((_BPW, _SEQ), jnp.int32),
    ],
)
def _sense_partials(x_hbm, wgp_hbm, ws2_hbm, out_hbm,
                    x_v, rows_v, acc_v, ws_v, part_v,
                    sem_ws, sem0, sem1, sem2, xr_v):
    wid = lax.axis_index("s") * _NC + lax.axis_index("c")
    sems = (sem0, sem1, sem2)
    zeros = jnp.zeros((_L,), jnp.float32)
    iota = lax.iota(jnp.int32, _L)

    pltpu.sync_copy(x_hbm.at[pl.ds(wid * _BPW, _BPW)], xr_v)
    for c in range(_SEQ):
        cv = jnp.full((_L,), c, jnp.int32)
        for j in range(_BPW // _L):
            rows = jnp.full((_L,), j * _L, jnp.int32) + iota
            x_v[c, pl.ds(j * _L, _L)] = plsc.load_gather(xr_v, [rows, cv])

    for q in range(2):
        pltpu.async_copy(ws2_hbm.at[x_v.at[0, pl.ds(q * _WSC, _WSC)]],
                         ws_v.at[q], sem_ws)

    def start_col(c, buf):
        pltpu.async_copy(wgp_hbm.at[x_v.at[c]], rows_v.at[buf], sems[buf])

    def wait_col(c, buf):
        pltpu.make_async_copy(
            wgp_hbm.at[x_v.at[c]], rows_v.at[buf], sems[buf]).wait()

    def acc_col(buf, first):
        def body(i, carry):
            for k in range(_KD):
                sl = pl.ds(k * _L, _L)
                v = rows_v[buf, i, sl]
                if first:
                    acc_v[i, sl] = v
                else:
                    plsc.addupdate(acc_v.at[i, sl], v)
            return carry
        lax.fori_loop(0, _BPW, body, 0, unroll=4)

    for t in range(_NBUF):
        start_col(2 + t, t)

    wait_col(2, 0)
    acc_col(0, first=True)
    start_col(5, 0)

    def ring_body(j, carry):
        c0 = 3 + 3 * j
        for t in range(3):
            buf = (1 + t) % _NBUF
            c = c0 + t
            wait_col(c, buf)
            acc_col(buf, first=False)

            @pl.when(c + _NBUF < _SEQ)
            def _():
                start_col(c + _NBUF, buf)
        return carry

    lax.fori_loop(0, 16, ring_body, 0)

    wait_col(51, 1)
    acc_col(1, first=False)

    accs = tuple(zeros for _ in range(_S))
    for q in range(_NQ):
        qb = q % 2
        idxref = x_v.at[0, pl.ds(q * _WSC, _WSC)]
        pltpu.make_async_copy(ws2_hbm.at[idxref], ws_v.at[qb], sem_ws).wait()

        def score_body(i, acc_c, q=q, qb=qb):
            ctx = [acc_v[q * _WSC + i, pl.ds(k * _L, _L)]
                   for k in range(_KD)]
            out = []
            for s in range(_S):
                a = acc_c[s]
                for k in range(_KD):
                    a = a + ws_v[qb, i, pl.ds(s * _D + k * _L, _L)] * ctx[k]
                out.append(a)
            return tuple(out)

        accs = lax.fori_loop(0, _WSC, score_body, accs)

        if q + 2 < _NQ:
            nidx = x_v.at[0, pl.ds((q + 2) * _WSC, _WSC)]
            pltpu.async_copy(ws2_hbm.at[nidx], ws_v.at[qb], sem_ws)

    for s in range(_S):
        for k in range(8):
            part_v[s, pl.ds(k * _L, _L)] = accs[s] if k == 0 else zeros
    pltpu.sync_copy(part_v, out_hbm.at[wid])


_PB = 128
_NPB = _VOCAB // _PB
_PTAIL = _VOCAB - _NPB * _PB
_KPB = -(-_NPB // _NW)


@functools.partial(
    pl.kernel,
    mesh=plsc.VectorSubcoreMesh(core_axis_name="c", subcore_axis_name="s"),
    compiler_params=pltpu.CompilerParams(use_tc_tiling_on_sc=True,
                                         needs_layout_passes=False,
                                         skip_device_barrier=True),
    out_type=jax.ShapeDtypeStruct((_VOCAB, 2 * _D), jnp.float32),
    scratch_types=[
        pltpu.VMEM((2, _PB, _D), jnp.float32),
        pltpu.VMEM((2, _PB, 2 * _D), jnp.float32),
        pltpu.SemaphoreType.DMA,
        pltpu.SemaphoreType.DMA,
        pltpu.SemaphoreType.DMA,
    ],
)
def _pad_wg_sc(wg_hbm, out_hbm, in_v, st_v, semi0, semi1, semo):
    wid = lax.axis_index("s") * _NC + lax.axis_index("c")
    semis = (semi0, semi1)

    def start_in(k, t):
        b = wid + _NW * k

        @pl.when(b < _NPB)
        def _():
            pltpu.async_copy(wg_hbm.at[pl.ds(b * _PB, _PB)],
                             in_v.at[t], semis[t])

    def step(k, t):
        b = wid + _NW * k

        @pl.when(b < _NPB)
        def _():
            pltpu.make_async_copy(wg_hbm.at[pl.ds(b * _PB, _PB)],
                                  in_v.at[t], semis[t]).wait()

            def rb(i, carry):
                for kk in range(_KD):
                    sl = pl.ds(kk * _L, _L)
                    st_v[t, i, sl] = in_v[t, i, sl]
                return carry

            lax.fori_loop(0, _PB, rb, 0, unroll=4)
            pltpu.async_copy(st_v.at[t], out_hbm.at[pl.ds(b * _PB, _PB)],
                             semo)
            pltpu.make_async_copy(st_v.at[t],
                                  out_hbm.at[pl.ds(b * _PB, _PB)],
                                  semo).wait()

    start_in(0, 0)
    start_in(1, 1)

    def loop(j, carry):
        for t in range(2):
            k = 2 * j + t
            step(k, t)
            start_in(k + 2, t)
        return carry

    lax.fori_loop(0, (_KPB + 1) // 2, loop, 0)

    @pl.when(wid == _NW - 1)
    def _():
        r0 = _NPB * _PB
        pltpu.sync_copy(wg_hbm.at[pl.ds(r0, _PTAIL)],
                        in_v.at[0, pl.ds(0, _PTAIL)])

        def rb(i, carry):
            for kk in range(_KD):
                sl = pl.ds(kk * _L, _L)
                st_v[0, i, sl] = in_v[0, i, sl]
            return carry

        lax.fori_loop(0, _PTAIL, rb, 0, unroll=4)
        pltpu.sync_copy(st_v.at[0, pl.ds(0, _PTAIL)],
                        out_hbm.at[pl.ds(r0, _PTAIL)])


@jax.jit
def kernel(x, W_g, W_s):
    wgp = _pad_wg_sc(W_g)
    ws2 = W_s.reshape(_VOCAB, _S * _D)
    partials = _sense_partials(x, wgp, ws2)
    return jax.nn.sigmoid(jnp.sum(partials, axis=(0, 2)))


# speedup vs baseline: 1.0900x; 1.0003x over previous
import functools

import jax
import jax.numpy as jnp
from jax import lax
from jax.experimental import pallas as pl
from jax.experimental.pallas import tpu as pltpu
from jax.experimental.pallas import tpu_sc as plsc

_VOCAB = 100000
_D = 64
_S = 8
_B = 4096
_SEQ = 52
_SEQP = 56
_L = 16
_NC = 2
_NS = 16
_NW = _NC * _NS
_BPW = _B // _NW
_KD = _D // _L
_NBUF = 3
_WSC = 32
_NQ = _BPW // _WSC


@functools.partial(
    pl.kernel,
    mesh=plsc.VectorSubcoreMesh(core_axis_name="c", subcore_axis_name="s"),
    compiler_params=pltpu.CompilerParams(use_tc_tiling_on_sc=True,
                                         needs_layout_passes=False,
                                         skip_device_barrier=True),
    out_type=jax.ShapeDtypeStruct((_NW, _S, 8 * _L), jnp.float32),
    scratch_types=[
        pltpu.VMEM((_SEQP, _BPW), jnp.int32),
        pltpu.VMEM((_NBUF, _BPW, 8 * _L), jnp.float32),
        pltpu.VMEM((_BPW, _D), jnp.float32),
        pltpu.VMEM((2, _WSC, _S * _D), jnp.float32),
        pltpu.VMEM((_S, 8 * _L), jnp.float32),
        pltpu.SemaphoreType.DMA,
        pltpu.SemaphoreType.DMA,
        pltpu.SemaphoreType.DMA,
        pltpu.SemaphoreType.DMA,
        pltpu.VMEM((_BPW, _SEQ), jnp.int32),
    ],
)
def _sense_partials(x_hbm, wgp_hbm, ws2_hbm, out_hbm,
                    x_v, rows_v, acc_v, ws_v, part_v,
                    sem_ws, sem0, sem1, sem2, xr_v):
    wid = lax.axis_index("s") * _NC + lax.axis_index("c")
    sems = (sem0, sem1, sem2)
    zeros = jnp.zeros((_L,), jnp.float32)
    iota = lax.iota(jnp.int32, _L)

    pltpu.sync_copy(x_hbm.at[pl.ds(wid * _BPW, _BPW)], xr_v)
    for c in range(_SEQ):
        cv = jnp.full((_L,), c, jnp.int32)
        for j in range(_BPW // _L):
            rows = jnp.full((_L,), j * _L, jnp.int32) + iota
            x_v[c, pl.ds(j * _L, _L)] = plsc.load_gather(xr_v, [rows, cv])

    for q in range(2):
        pltpu.async_copy(ws2_hbm.at[x_v.at[0, pl.ds(q * _WSC, _WSC)]],
                         ws_v.at[q], sem_ws)

    def start_col(c, buf):
        pltpu.async_copy(wgp_hbm.at[x_v.at[c]], rows_v.at[buf], sems[buf])

    def wait_col(c, buf):
        pltpu.make_async_copy(
            wgp_hbm.at[x_v.at[c]], rows_v.at[buf], sems[buf]).wait()

    def acc_col(buf, first):
        def body(i, carry):
            for k in range(_KD):
                sl = pl.ds(k * _L, _L)
                v = rows_v[buf, i, sl]
                if first:
                    acc_v[i, sl] = v
                else:
                    plsc.addupdate(acc_v.at[i, sl], v)
            return carry
        lax.fori_loop(0, _BPW, body, 0, unroll=4)

    for t in range(_NBUF):
        start_col(2 + t, t)

    wait_col(2, 0)
    acc_col(0, first=True)
    start_col(5, 0)

    def ring_body(j, carry):
        c0 = 3 + 3 * j
        for t in range(3):
            buf = (1 + t) % _NBUF
            c = c0 + t
            wait_col(c, buf)
            acc_col(buf, first=False)

            @pl.when(c + _NBUF < _SEQ)
            def _():
                start_col(c + _NBUF, buf)
        return carry

    lax.fori_loop(0, 16, ring_body, 0)

    wait_col(51, 1)
    acc_col(1, first=False)

    accs = tuple(zeros for _ in range(_S))
    for q in range(_NQ):
        qb = q % 2
        idxref = x_v.at[0, pl.ds(q * _WSC, _WSC)]
        pltpu.make_async_copy(ws2_hbm.at[idxref], ws_v.at[qb], sem_ws).wait()

        def score_body(i, acc_c, q=q, qb=qb):
            ctx = [acc_v[q * _WSC + i, pl.ds(k * _L, _L)]
                   for k in range(_KD)]
            out = []
            for s in range(_S):
                a = acc_c[s]
                for k in range(_KD):
                    a = a + ws_v[qb, i, pl.ds(s * _D + k * _L, _L)] * ctx[k]
                out.append(a)
            return tuple(out)

        accs = lax.fori_loop(0, _WSC, score_body, accs)

        if q + 2 < _NQ:
            nidx = x_v.at[0, pl.ds((q + 2) * _WSC, _WSC)]
            pltpu.async_copy(ws2_hbm.at[nidx], ws_v.at[qb], sem_ws)

    for s in range(_S):
        for k in range(8):
            part_v[s, pl.ds(k * _L, _L)] = accs[s] if k == 0 else zeros
    pltpu.sync_copy(part_v, out_hbm.at[wid])


_PB = 128
_NPB = _VOCAB // _PB
_PTAIL = _VOCAB - _NPB * _PB
_KPB = -(-_NPB // _NW)


@functools.partial(
    pl.kernel,
    mesh=plsc.VectorSubcoreMesh(core_axis_name="c", subcore_axis_name="s"),
    compiler_params=pltpu.CompilerParams(use_tc_tiling_on_sc=True),
    out_type=jax.ShapeDtypeStruct((_VOCAB, 2 * _D), jnp.float32),
    scratch_types=[
        pltpu.VMEM((2, _PB, _D), jnp.float32),
        pltpu.VMEM((2, _PB, 2 * _D), jnp.float32),
        pltpu.SemaphoreType.DMA,
        pltpu.SemaphoreType.DMA,
        pltpu.SemaphoreType.DMA,
    ],
)
def _pad_wg_sc(wg_hbm, out_hbm, in_v, st_v, semi0, semi1, semo):
    wid = lax.axis_index("s") * _NC + lax.axis_index("c")
    semis = (semi0, semi1)

    def start_in(k, t):
        b = wid + _NW * k

        @pl.when(b < _NPB)
        def _():
            pltpu.async_copy(wg_hbm.at[pl.ds(b * _PB, _PB)],
                             in_v.at[t], semis[t])

    def step(k, t):
        b = wid + _NW * k

        @pl.when(b < _NPB)
        def _():
            pltpu.make_async_copy(wg_hbm.at[pl.ds(b * _PB, _PB)],
                                  in_v.at[t], semis[t]).wait()

            def rb(i, carry):
                for kk in range(_KD):
                    sl = pl.ds(kk * _L, _L)
                    st_v[t, i, sl] = in_v[t, i, sl]
                return carry

            lax.fori_loop(0, _PB, rb, 0, unroll=4)
            pltpu.async_copy(st_v.at[t], out_hbm.at[pl.ds(b * _PB, _PB)],
                             semo)
            pltpu.make_async_copy(st_v.at[t],
                                  out_hbm.at[pl.ds(b * _PB, _PB)],
                                  semo).wait()

    start_in(0, 0)
    start_in(1, 1)

    def loop(j, carry):
        for t in range(2):
            k = 2 * j + t
            step(k, t)
            start_in(k + 2, t)
        return carry

    lax.fori_loop(0, (_KPB + 1) // 2, loop, 0)

    @pl.when(wid == _NW - 1)
    def _():
        r0 = _NPB * _PB
        pltpu.sync_copy(wg_hbm.at[pl.ds(r0, _PTAIL)],
                        in_v.at[0, pl.ds(0, _PTAIL)])

        def rb(i, carry):
            for kk in range(_KD):
                sl = pl.ds(kk * _L, _L)
                st_v[0, i, sl] = in_v[0, i, sl]
            return carry

        lax.fori_loop(0, _PTAIL, rb, 0, unroll=4)
        pltpu.sync_copy(st_v.at[0, pl.ds(0, _PTAIL)],
                        out_hbm.at[pl.ds(r0, _PTAIL)])


@jax.jit
def kernel(x, W_g, W_s):
    wgp = _pad_wg_sc(W_g)
    ws2 = W_s.reshape(_VOCAB, _S * _D)
    partials = _sense_partials(x, wgp, ws2)
    return jax.nn.sigmoid(jnp.sum(partials, axis=(0, 2)))


# speedup vs baseline: 1.2657x; 1.1612x over previous
import functools

import jax
import jax.numpy as jnp
from jax import lax
from jax.experimental import pallas as pl
from jax.experimental.pallas import tpu as pltpu
from jax.experimental.pallas import tpu_sc as plsc

_VOCAB = 100000
_D = 64
_S = 8
_B = 4096
_SEQ = 52
_L = 16
_NC = 2
_NS = 16
_NW = _NC * _NS
_BPW = _B // _NW
_KD = _D // _L
_NBUF = 3
_WSC = 32
_NQ = _BPW // _WSC


@functools.partial(
    pl.kernel,
    mesh=plsc.VectorSubcoreMesh(core_axis_name="c", subcore_axis_name="s"),
    compiler_params=pltpu.CompilerParams(use_tc_tiling_on_sc=False,
                                         needs_layout_passes=False),
    out_type=(jax.ShapeDtypeStruct((_NW, _BPW, 2 * _D), jnp.float32),
              jax.ShapeDtypeStruct((_NW, _BPW), jnp.int32)),
    scratch_types=[
        pltpu.VMEM((_BPW, _SEQ), jnp.int32),
        pltpu.VMEM((_SEQ, _BPW), jnp.int32),
        pltpu.VMEM((_NBUF, _BPW, _D), jnp.float32),
        pltpu.VMEM((_BPW, 2 * _D), jnp.float32),
        pltpu.SemaphoreType.DMA,
        pltpu.SemaphoreType.DMA,
        pltpu.SemaphoreType.DMA,
    ],
)
def _context_sums(x_hbm, wg_hbm, acc_hbm, xw_hbm,
                  xr_v, x_v, rows_v, acc_v, sem0, sem1, sem2):
    wid = lax.axis_index("s") * _NC + lax.axis_index("c")
    sems = (sem0, sem1, sem2)
    iota = lax.iota(jnp.int32, _L)
    zeros = jnp.zeros((_L,), jnp.float32)

    pltpu.sync_copy(x_hbm.at[pl.ds(wid * _BPW, _BPW)], xr_v)

    for c in range(_SEQ):
        cv = jnp.full((_L,), c, jnp.int32)
        for j in range(_BPW // _L):
            rows = jnp.full((_L,), j * _L, jnp.int32) + iota
            x_v[c, pl.ds(j * _L, _L)] = plsc.load_gather(xr_v, [rows, cv])

    pltpu.sync_copy(x_v.at[0], xw_hbm.at[wid])

    def start_col(c, buf):
        pltpu.async_copy(wg_hbm.at[x_v.at[c]], rows_v.at[buf], sems[buf])

    def wait_col(c, buf):
        pltpu.make_async_copy(
            wg_hbm.at[x_v.at[c]], rows_v.at[buf], sems[buf]).wait()

    def acc_col(buf, first):
        def body(i, carry):
            for k in range(_KD):
                sl = pl.ds(k * _L, _L)
                v = rows_v[buf, i, sl]
                if first:
                    acc_v[i, sl] = v
                else:
                    plsc.addupdate(acc_v.at[i, sl], v)
            if first:
                for k in range(_KD, 2 * _KD):
                    acc_v[i, pl.ds(k * _L, _L)] = zeros
            return carry
        lax.fori_loop(0, _BPW, body, 0, unroll=4)

    for t in range(_NBUF):
        start_col(2 + t, t)

    wait_col(2, 0)
    acc_col(0, first=True)
    start_col(5, 0)

    def ring_body(j, carry):
        c0 = 3 + 3 * j
        for t in range(3):
            buf = (1 + t) % _NBUF
            c = c0 + t
            wait_col(c, buf)
            acc_col(buf, first=False)

            @pl.when(c + _NBUF < _SEQ)
            def _():
                start_col(c + _NBUF, buf)
        return carry

    lax.fori_loop(0, 16, ring_body, 0)

    wait_col(51, 1)
    acc_col(1, first=False)

    pltpu.sync_copy(acc_v, acc_hbm.at[wid])


@functools.partial(
    pl.kernel,
    mesh=plsc.VectorSubcoreMesh(core_axis_name="c", subcore_axis_name="s"),
    compiler_params=pltpu.CompilerParams(use_tc_tiling_on_sc=True,
                                         needs_layout_passes=False),
    out_type=jax.ShapeDtypeStruct((_NW, _S, 8 * _L), jnp.float32),
    scratch_types=[
        pltpu.VMEM((_BPW,), jnp.int32),
        pltpu.VMEM((_BPW, 2 * _D), jnp.float32),
        pltpu.VMEM((2, _WSC, _S * _D), jnp.float32),
        pltpu.VMEM((_S, 8 * _L), jnp.float32),
        pltpu.SemaphoreType.DMA,
    ],
)
def _sense_scores(xw_hbm, acc_hbm, ws2_hbm, out_hbm,
                  x0_v, accb_v, ws_v, part_v, sem_ws):
    wid = lax.axis_index("s") * _NC + lax.axis_index("c")
    zeros = jnp.zeros((_L,), jnp.float32)

    pltpu.sync_copy(xw_hbm.at[wid], x0_v)
    for q in range(2):
        pltpu.async_copy(ws2_hbm.at[x0_v.at[pl.ds(q * _WSC, _WSC)]],
                         ws_v.at[q], sem_ws)
    pltpu.sync_copy(acc_hbm.at[wid], accb_v)

    accs = tuple(zeros for _ in range(_S))
    for q in range(_NQ):
        qb = q % 2
        idxref = x0_v.at[pl.ds(q * _WSC, _WSC)]
        pltpu.make_async_copy(ws2_hbm.at[idxref], ws_v.at[qb], sem_ws).wait()

        def score_body(i, acc_c, q=q, qb=qb):
            ctx = [accb_v[q * _WSC + i, pl.ds(k * _L, _L)]
                   for k in range(_KD)]
            out = []
            for s in range(_S):
                a = acc_c[s]
                for k in range(_KD):
                    a = a + ws_v[qb, i, pl.ds(s * _D + k * _L, _L)] * ctx[k]
                out.append(a)
            return tuple(out)

        accs = lax.fori_loop(0, _WSC, score_body, accs)

        if q + 2 < _NQ:
            nidx = x0_v.at[pl.ds((q + 2) * _WSC, _WSC)]
            pltpu.async_copy(ws2_hbm.at[nidx], ws_v.at[qb], sem_ws)

    for s in range(_S):
        for k in range(8):
            part_v[s, pl.ds(k * _L, _L)] = accs[s] if k == 0 else zeros
    pltpu.sync_copy(part_v, out_hbm.at[wid])


@jax.jit
def kernel(x, W_g, W_s):
    ws2 = W_s.reshape(_VOCAB, _S * _D)
    acc, xw = _context_sums(x, W_g)
    partials = _sense_scores(xw, acc, ws2)
    return jax.nn.sigmoid(jnp.sum(partials, axis=(0, 2)))
